# baseline (speedup 1.0000x reference)
import functools

import jax
import jax.numpy as jnp
from jax import lax
from jax.experimental import pallas as pl
from jax.experimental.pallas import tpu as pltpu
from jax.experimental.pallas import tpu_sc as plsc

_N = 10000
_DEG = 32
_FEAT = 128
_NW = 32
_NPW = 320
_NPAD = _NW * _NPW
_EPB = 64
_GPB = _EPB // _DEG
_NBLK = _NPW // _GPB

_mesh = plsc.VectorSubcoreMesh(core_axis_name="c", subcore_axis_name="s")
_cparams = pltpu.CompilerParams(needs_layout_passes=False)


def _rsqrt_vec(v):
    bits = plsc.bitcast(v, jnp.int32)
    y = plsc.bitcast(jnp.int32(0x5F3759DF) - (bits >> 1), jnp.float32)
    for _ in range(3):
        y = y * (1.5 - 0.5 * v * y * y)
    return y


def _splat(s, dtype=jnp.float32):
    return jnp.full((16,), s, dtype)


@functools.partial(
    pl.kernel,
    out_type=jax.ShapeDtypeStruct((_NPAD, _FEAT), jnp.float32),
    mesh=_mesh,
    compiler_params=_cparams,
    scratch_types=[
        pltpu.VMEM((3, _EPB), jnp.int32),
        pltpu.VMEM((_NPAD,), jnp.float32),
        pltpu.VMEM((16,), jnp.float32),
        pltpu.VMEM((_EPB,), jnp.float32),
        pltpu.VMEM((2, _GPB, _FEAT), jnp.float32),
        pltpu.VMEM((2, _EPB, _FEAT), jnp.float32),
        pltpu.VMEM((_GPB, _FEAT), jnp.float32),
        pltpu.VMEM_SHARED((_NPAD, _FEAT), jnp.float32),
        pltpu.VMEM_SHARED((_NPAD,), jnp.float32),
        pltpu.SemaphoreType.DMA,
        pltpu.SemaphoreType.DMA,
        pltpu.SemaphoreType.DMA,
        pltpu.SemaphoreType.DMA,
    ],
)
def _agnn(x_hbm, col_hbm, beta_hbm, out_hbm,
          colv3, invv, betab, invchunk, xi, rows, outb, shx, shinv,
          gsem, osem, csem, xsem):
    sid = lax.axis_index("s")
    wid = sid * 2 + lax.axis_index("c")
    pltpu.sync_copy(beta_hbm, betab)
    beta = betab[...][0]
    lanes = lax.iota(jnp.int32, 16)
    stripe = _NPAD // 16

    nchunks = stripe // _EPB
    pltpu.async_copy(x_hbm.at[pl.ds(sid * stripe, _EPB)], rows.at[0], gsem)

    def stage_pair(ci, carry):
        for pp in range(2):
            c = ci * 2 + pp
            row0 = sid * stripe + c * _EPB
            pltpu.make_async_copy(x_hbm.at[pl.ds(row0, _EPB)], rows.at[pp],
                                  gsem).wait()
            if pp == 1:
                pltpu.make_async_copy(
                    rows.at[0], shx.at[pl.ds(row0 - _EPB, _EPB)],
                    osem).wait()
            else:
                @pl.when(ci > 0)
                def _():
                    pltpu.make_async_copy(
                        rows.at[1], shx.at[pl.ds(row0 - _EPB, _EPB)],
                        osem).wait()
            if pp == 0:
                pltpu.async_copy(x_hbm.at[pl.ds(row0 + _EPB, _EPB)],
                                 rows.at[1], gsem)
            else:
                @pl.when(ci < nchunks // 2 - 1)
                def _():
                    pltpu.async_copy(x_hbm.at[pl.ds(row0 + _EPB, _EPB)],
                                     rows.at[0], gsem)
            pltpu.async_copy(rows.at[pp], shx.at[pl.ds(row0, _EPB)], osem)
            for q in range(_EPB // 16):
                ss = jnp.zeros((16,), jnp.float32)
                for i in range(16):
                    v = rows[pp, q * 16 + i, pl.ds(0, 16)]
                    acc = v * v
                    for r in range(1, 8):
                        v = rows[pp, q * 16 + i, pl.ds(r * 16, 16)]
                        acc = acc + v * v
                    ss = jnp.where(lanes == i, _splat(jnp.sum(acc)), ss)
                norm = ss * _rsqrt_vec(ss)
                invchunk[pl.ds(q * 16, 16)] = 1.0 / (norm + 1e-12)
            pltpu.sync_copy(invchunk, shinv.at[pl.ds(row0, _EPB)])
        return carry

    lax.fori_loop(0, nchunks // 2, stage_pair, 0)
    last_row0 = sid * stripe + (nchunks - 1) * _EPB
    pltpu.make_async_copy(rows.at[1], shx.at[pl.ds(last_row0, _EPB)],
                          osem).wait()
    plsc.subcore_barrier()
    pltpu.sync_copy(shinv, invv)
    pltpu.sync_copy(col_hbm.at[wid, 0], colv3.at[0])
    pltpu.async_copy(shx.at[colv3.at[0]], rows.at[0], gsem)
    pltpu.async_copy(col_hbm.at[wid, 1], colv3.at[1], csem)
    pltpu.async_copy(shx.at[pl.ds(wid * _NPW, _GPB)], xi.at[0], xsem)

    def block_pair(bb, carry):
        for p in range(2):
            b = bb * 2 + p
            node0 = wid * _NPW + b * _GPB
            i3 = lax.rem(b, 3)
            pltpu.make_async_copy(shx.at[pl.ds(node0, _GPB)], xi.at[p],
                                  xsem).wait()
            pltpu.make_async_copy(shx.at[colv3.at[i3]], rows.at[p],
                                  gsem).wait()

            def next_dmas():
                i3n = lax.rem(b + 1, 3)
                pltpu.make_async_copy(
                    col_hbm.at[wid, b + 1], colv3.at[i3n], csem).wait()

                @pl.when(b + 2 < _NBLK)
                def _():
                    pltpu.async_copy(
                        col_hbm.at[wid, b + 2], colv3.at[lax.rem(b + 2, 3)],
                        csem)
                pltpu.async_copy(shx.at[colv3.at[i3n]], rows.at[1 - p], gsem)
                pltpu.async_copy(shx.at[pl.ds(node0 + _GPB, _GPB)],
                                 xi.at[1 - p], xsem)

            if p == 0:
                next_dmas()
            else:
                @pl.when(b < _NBLK - 1)
                def _():
                    next_dmas()
            if p == 1:
                pltpu.make_async_copy(
                    outb, out_hbm.at[pl.ds(node0 - _GPB, _GPB)], osem).wait()
            else:
                @pl.when(b > 0)
                def _():
                    pltpu.make_async_copy(
                        outb, out_hbm.at[pl.ds(node0 - _GPB, _GPB)],
                        osem).wait()

            def node(n, c2):
                nb = n * _DEG
                xr = [xi[p, n, pl.ds(r * 16, 16)] for r in range(8)]
                bi = plsc.load_gather(invv, [_splat(node0 + n, jnp.int32)])
                bib = bi * beta
                abv = _splat(jnp.abs(beta))
                denv = jnp.zeros((16,), jnp.float32)
                for g in range(2):
                    jidx = colv3[i3, pl.ds(nb + g * 16, 16)]
                    invj = plsc.load_gather(invv, [jidx])
                    for jj in range(16):
                        rr = nb + g * 16 + jj
                        rv = [rows[p, rr, pl.ds(r * 16, 16)]
                              for r in range(8)]
                        acc = xr[0] * rv[0]
                        for r in range(1, 8):
                            acc = acc + xr[r] * rv[r]
                        sv = _splat(jnp.sum(acc)) * _splat(invj[jj]) * bib
                        w = jnp.exp(sv - abv)
                        denv = denv + w
                        if g == 0 and jj == 0:
                            for r in range(8):
                                outb[n, pl.ds(r * 16, 16)] = w * rv[r]
                        else:
                            for r in range(8):
                                plsc.addupdate(
                                    outb.at[n, pl.ds(r * 16, 16)], w * rv[r])
                rden = 1.0 / denv
                for r in range(8):
                    outb[n, pl.ds(r * 16, 16)] = (
                        outb[n, pl.ds(r * 16, 16)] * rden)
                return c2

            lax.fori_loop(0, _GPB, node, 0)
            pltpu.async_copy(outb, out_hbm.at[pl.ds(node0, _GPB)], osem)
        return carry

    lax.fori_loop(0, _NBLK // 2, block_pair, 0)
    last0 = wid * _NPW + (_NBLK - 1) * _GPB
    pltpu.make_async_copy(outb, out_hbm.at[pl.ds(last0, _GPB)], osem).wait()


def kernel(x, row_id, row_ptr, col_id, beta):
    del row_id, row_ptr
    xp = jnp.zeros((_NPAD, _FEAT), jnp.float32).at[:_N].set(x)
    colp = jnp.zeros((_NPAD * _DEG,), jnp.int32).at[: _N * _DEG].set(col_id)
    col3 = colp.reshape(_NW, _NBLK, _EPB)
    beta16 = jnp.zeros((16,), jnp.float32).at[0].set(beta[0])
    out = _agnn(xp, col3, beta16)
    return out[:_N]


# speedup vs baseline: 3.6091x; 3.6091x over previous
import functools

import jax
import jax.numpy as jnp
from jax import lax
from jax.experimental import pallas as pl
from jax.experimental.pallas import tpu as pltpu
from jax.experimental.pallas import tpu_sc as plsc

_N = 10000
_DEG = 32
_FEAT = 128
_NW = 32
_NPW = 320
_NPAD = _NW * _NPW
_EPB = 64
_GPB = _EPB // _DEG
_NBLK = _NPW // _GPB

_mesh = plsc.VectorSubcoreMesh(core_axis_name="c", subcore_axis_name="s")
_cparams = pltpu.CompilerParams(needs_layout_passes=False)


def _rsqrt_vec(v):
    bits = plsc.bitcast(v, jnp.int32)
    y = plsc.bitcast(jnp.int32(0x5F3759DF) - (bits >> 1), jnp.float32)
    for _ in range(3):
        y = y * (1.5 - 0.5 * v * y * y)
    return y


def _splat(s, dtype=jnp.float32):
    return jnp.full((16,), s, dtype)


@functools.partial(
    pl.kernel,
    out_type=jax.ShapeDtypeStruct((_NPAD, _FEAT), jnp.float32),
    mesh=_mesh,
    compiler_params=_cparams,
    scratch_types=[
        pltpu.VMEM((3, _EPB), jnp.int32),
        pltpu.VMEM((_NPAD,), jnp.float32),
        pltpu.VMEM((16,), jnp.float32),
        pltpu.VMEM((_EPB,), jnp.float32),
        pltpu.VMEM((2, _GPB, _FEAT), jnp.float32),
        pltpu.VMEM((2, _EPB, _FEAT), jnp.float32),
        pltpu.VMEM((_GPB, _FEAT), jnp.float32),
        pltpu.VMEM_SHARED((_NPAD, _FEAT), jnp.float32),
        pltpu.VMEM_SHARED((_NPAD,), jnp.float32),
        pltpu.SemaphoreType.DMA,
        pltpu.SemaphoreType.DMA,
        pltpu.SemaphoreType.DMA,
        pltpu.SemaphoreType.DMA,
    ],
)
def _agnn(x_hbm, col_hbm, beta_hbm, out_hbm,
          colv3, invv, betab, invchunk, xi, rows, outb, shx, shinv,
          gsem, osem, csem, xsem):
    sid = lax.axis_index("s")
    wid = sid * 2 + lax.axis_index("c")
    pltpu.sync_copy(beta_hbm, betab)
    beta = betab[...][0]
    lanes = lax.iota(jnp.int32, 16)
    stripe = _NPAD // 16

    nchunks = stripe // _EPB
    pltpu.async_copy(x_hbm.at[pl.ds(sid * stripe, _EPB)], rows.at[0], gsem)

    def stage_pair(ci, carry):
        for pp in range(2):
            c = ci * 2 + pp
            row0 = sid * stripe + c * _EPB
            pltpu.make_async_copy(x_hbm.at[pl.ds(row0, _EPB)], rows.at[pp],
                                  gsem).wait()
            if pp == 1:
                pltpu.make_async_copy(
                    rows.at[0], shx.at[pl.ds(row0 - _EPB, _EPB)],
                    osem).wait()
            else:
                @pl.when(ci > 0)
                def _():
                    pltpu.make_async_copy(
                        rows.at[1], shx.at[pl.ds(row0 - _EPB, _EPB)],
                        osem).wait()
            if pp == 0:
                pltpu.async_copy(x_hbm.at[pl.ds(row0 + _EPB, _EPB)],
                                 rows.at[1], gsem)
            else:
                @pl.when(ci < nchunks // 2 - 1)
                def _():
                    pltpu.async_copy(x_hbm.at[pl.ds(row0 + _EPB, _EPB)],
                                     rows.at[0], gsem)
            pltpu.async_copy(rows.at[pp], shx.at[pl.ds(row0, _EPB)], osem)
            for q in range(_EPB // 16):
                ss = jnp.zeros((16,), jnp.float32)
                for i in range(16):
                    v = rows[pp, q * 16 + i, pl.ds(0, 16)]
                    acc = v * v
                    for r in range(1, 8):
                        v = rows[pp, q * 16 + i, pl.ds(r * 16, 16)]
                        acc = acc + v * v
                    ss = jnp.where(lanes == i, _splat(jnp.sum(acc)), ss)
                norm = ss * _rsqrt_vec(ss)
                invchunk[pl.ds(q * 16, 16)] = 1.0 / (norm + 1e-12)
            pltpu.sync_copy(invchunk, shinv.at[pl.ds(row0, _EPB)])
        return carry

    lax.fori_loop(0, nchunks // 2, stage_pair, 0)
    last_row0 = sid * stripe + (nchunks - 1) * _EPB
    pltpu.make_async_copy(rows.at[1], shx.at[pl.ds(last_row0, _EPB)],
                          osem).wait()
    plsc.subcore_barrier()
    pltpu.sync_copy(shinv, invv)
    pltpu.sync_copy(col_hbm.at[wid, 0], colv3.at[0])
    pltpu.async_copy(shx.at[colv3.at[0]], rows.at[0], gsem)
    pltpu.async_copy(col_hbm.at[wid, 1], colv3.at[1], csem)
    pltpu.async_copy(shx.at[pl.ds(wid * _NPW, _GPB)], xi.at[0], xsem)

    def block_pair(bb, carry):
        for p in range(2):
            b = bb * 2 + p
            node0 = wid * _NPW + b * _GPB
            i3 = lax.rem(b, 3)
            pltpu.make_async_copy(shx.at[pl.ds(node0, _GPB)], xi.at[p],
                                  xsem).wait()
            pltpu.make_async_copy(shx.at[colv3.at[i3]], rows.at[p],
                                  gsem).wait()

            def next_dmas():
                i3n = lax.rem(b + 1, 3)
                pltpu.make_async_copy(
                    col_hbm.at[wid, b + 1], colv3.at[i3n], csem).wait()

                @pl.when(b + 2 < _NBLK)
                def _():
                    pltpu.async_copy(
                        col_hbm.at[wid, b + 2], colv3.at[lax.rem(b + 2, 3)],
                        csem)
                pltpu.async_copy(shx.at[colv3.at[i3n]], rows.at[1 - p], gsem)
                pltpu.async_copy(shx.at[pl.ds(node0 + _GPB, _GPB)],
                                 xi.at[1 - p], xsem)

            if p == 0:
                next_dmas()
            else:
                @pl.when(b < _NBLK - 1)
                def _():
                    next_dmas()
            if p == 1:
                pltpu.make_async_copy(
                    outb, out_hbm.at[pl.ds(node0 - _GPB, _GPB)], osem).wait()
            else:
                @pl.when(b > 0)
                def _():
                    pltpu.make_async_copy(
                        outb, out_hbm.at[pl.ds(node0 - _GPB, _GPB)],
                        osem).wait()

            def node(n, c2):
                nb = n * _DEG
                xr = [xi[p, n, pl.ds(r * 16, 16)] for r in range(8)]
                bi = plsc.load_gather(invv, [_splat(node0 + n, jnp.int32)])
                bib = bi * beta
                abv = _splat(jnp.abs(beta))
                denv = jnp.zeros((16,), jnp.float32)
                ao = [jnp.zeros((16,), jnp.float32) for _ in range(8)]
                for g in range(2):
                    jidx = colv3[i3, pl.ds(nb + g * 16, 16)]
                    invj = plsc.load_gather(invv, [jidx])
                    for jj in range(16):
                        rr = nb + g * 16 + jj
                        rv = [rows[p, rr, pl.ds(r * 16, 16)]
                              for r in range(8)]
                        acc = xr[0] * rv[0]
                        for r in range(1, 8):
                            acc = acc + xr[r] * rv[r]
                        sv = _splat(jnp.sum(acc)) * _splat(invj[jj]) * bib
                        w = jnp.exp(sv - abv)
                        denv = denv + w
                        for r in range(8):
                            ao[r] = ao[r] + w * rv[r]
                rden = 1.0 / denv
                for r in range(8):
                    outb[n, pl.ds(r * 16, 16)] = ao[r] * rden
                return c2

            lax.fori_loop(0, _GPB, node, 0)
            pltpu.async_copy(outb, out_hbm.at[pl.ds(node0, _GPB)], osem)
        return carry

    lax.fori_loop(0, _NBLK // 2, block_pair, 0)
    last0 = wid * _NPW + (_NBLK - 1) * _GPB
    pltpu.make_async_copy(outb, out_hbm.at[pl.ds(last0, _GPB)], osem).wait()


def kernel(x, row_id, row_ptr, col_id, beta):
    del row_id, row_ptr
    xp = jnp.zeros((_NPAD, _FEAT), jnp.float32).at[:_N].set(x)
    colp = jnp.zeros((_NPAD * _DEG,), jnp.int32).at[: _N * _DEG].set(col_id)
    col3 = colp.reshape(_NW, _NBLK, _EPB)
    beta16 = jnp.zeros((16,), jnp.float32).at[0].set(beta[0])
    out = _agnn(xp, col3, beta16)
    return out[:_N]


# speedup vs baseline: 3.8213x; 1.0588x over previous
import functools

import jax
import jax.numpy as jnp
from jax import lax
from jax.experimental import pallas as pl
from jax.experimental.pallas import tpu as pltpu
from jax.experimental.pallas import tpu_sc as plsc

_N = 10000
_DEG = 32
_FEAT = 128
_NW = 32
_NPW = 320
_NPAD = _NW * _NPW
_EPB = 64
_GPB = _EPB // _DEG
_NBLK = _NPW // _GPB

_mesh = plsc.VectorSubcoreMesh(core_axis_name="c", subcore_axis_name="s")
_cparams = pltpu.CompilerParams(needs_layout_passes=False)


def _rsqrt_vec(v):
    bits = plsc.bitcast(v, jnp.int32)
    y = plsc.bitcast(jnp.int32(0x5F3759DF) - (bits >> 1), jnp.float32)
    for _ in range(3):
        y = y * (1.5 - 0.5 * v * y * y)
    return y


def _splat(s, dtype=jnp.float32):
    return jnp.full((16,), s, dtype)


@functools.partial(
    pl.kernel,
    out_type=jax.ShapeDtypeStruct((_N, _FEAT), jnp.float32),
    mesh=_mesh,
    compiler_params=_cparams,
    scratch_types=[
        pltpu.VMEM((3, _EPB), jnp.int32),
        pltpu.VMEM((_NPAD,), jnp.float32),
        pltpu.VMEM((16,), jnp.float32),
        pltpu.VMEM((_EPB,), jnp.float32),
        pltpu.VMEM((2, _GPB, _FEAT), jnp.float32),
        pltpu.VMEM((2, _EPB, _FEAT), jnp.float32),
        pltpu.VMEM((_GPB, _FEAT), jnp.float32),
        pltpu.VMEM_SHARED((_NPAD, _FEAT), jnp.float32),
        pltpu.VMEM_SHARED((_NPAD,), jnp.float32),
        pltpu.SemaphoreType.DMA,
        pltpu.SemaphoreType.DMA,
        pltpu.SemaphoreType.DMA,
        pltpu.SemaphoreType.DMA,
    ],
)
def _agnn(x_hbm, col_hbm, beta_hbm, out_hbm,
          colv3, invv, betab, invchunk, xi, rows, outb, shx, shinv,
          gsem, osem, csem, xsem):
    sid = lax.axis_index("s")
    wid = sid * 2 + lax.axis_index("c")
    pltpu.sync_copy(beta_hbm, betab)
    beta = betab[...][0]
    lanes = lax.iota(jnp.int32, 16)
    stripe = _NPAD // 16

    nchunks = stripe // _EPB
    pltpu.async_copy(
        x_hbm.at[pl.ds(jnp.minimum(sid * stripe, _N - _EPB), _EPB)],
        rows.at[0], gsem)

    def stage_pair(ci, carry):
        for pp in range(2):
            c = ci * 2 + pp
            row0 = jnp.minimum(sid * stripe + c * _EPB, _N - _EPB)
            pltpu.make_async_copy(x_hbm.at[pl.ds(row0, _EPB)], rows.at[pp],
                                  gsem).wait()
            if pp == 1:
                pltpu.make_async_copy(
                    rows.at[0], shx.at[pl.ds(row0 - _EPB, _EPB)],
                    osem).wait()
            else:
                @pl.when(ci > 0)
                def _():
                    pltpu.make_async_copy(
                        rows.at[1], shx.at[pl.ds(row0 - _EPB, _EPB)],
                        osem).wait()
            nrow0 = jnp.minimum(sid * stripe + (c + 1) * _EPB, _N - _EPB)
            if pp == 0:
                pltpu.async_copy(x_hbm.at[pl.ds(nrow0, _EPB)],
                                 rows.at[1], gsem)
            else:
                @pl.when(ci < nchunks // 2 - 1)
                def _():
                    pltpu.async_copy(x_hbm.at[pl.ds(nrow0, _EPB)],
                                     rows.at[0], gsem)
            pltpu.async_copy(rows.at[pp], shx.at[pl.ds(row0, _EPB)], osem)
            for q in range(_EPB // 16):
                ss = jnp.zeros((16,), jnp.float32)
                for i in range(16):
                    v = rows[pp, q * 16 + i, pl.ds(0, 16)]
                    acc = v * v
                    for r in range(1, 8):
                        v = rows[pp, q * 16 + i, pl.ds(r * 16, 16)]
                        acc = acc + v * v
                    ss = jnp.where(lanes == i, _splat(jnp.sum(acc)), ss)
                norm = ss * _rsqrt_vec(ss)
                invchunk[pl.ds(q * 16, 16)] = 1.0 / (norm + 1e-12)
            pltpu.sync_copy(invchunk, shinv.at[pl.ds(row0, _EPB)])
        return carry

    lax.fori_loop(0, nchunks // 2, stage_pair, 0)
    last_row0 = sid * stripe + (nchunks - 1) * _EPB
    pltpu.make_async_copy(rows.at[1], shx.at[pl.ds(last_row0, _EPB)],
                          osem).wait()
    plsc.subcore_barrier()
    pltpu.sync_copy(shinv, invv)
    pltpu.sync_copy(col_hbm.at[wid, 0], colv3.at[0])
    pltpu.async_copy(shx.at[colv3.at[0]], rows.at[0], gsem)
    pltpu.async_copy(col_hbm.at[wid, 1], colv3.at[1], csem)
    pltpu.async_copy(shx.at[pl.ds(wid * _NPW, _GPB)], xi.at[0], xsem)

    def block_pair(bb, carry):
        for p in range(2):
            b = bb * 2 + p
            node0 = wid * _NPW + b * _GPB
            i3 = lax.rem(b, 3)
            pltpu.make_async_copy(shx.at[pl.ds(node0, _GPB)], xi.at[p],
                                  xsem).wait()
            pltpu.make_async_copy(shx.at[colv3.at[i3]], rows.at[p],
                                  gsem).wait()

            def next_dmas():
                i3n = lax.rem(b + 1, 3)
                pltpu.make_async_copy(
                    col_hbm.at[wid, b + 1], colv3.at[i3n], csem).wait()

                @pl.when(b + 2 < _NBLK)
                def _():
                    pltpu.async_copy(
                        col_hbm.at[wid, b + 2], colv3.at[lax.rem(b + 2, 3)],
                        csem)
                pltpu.async_copy(shx.at[colv3.at[i3n]], rows.at[1 - p], gsem)
                pltpu.async_copy(shx.at[pl.ds(node0 + _GPB, _GPB)],
                                 xi.at[1 - p], xsem)

            if p == 0:
                next_dmas()
            else:
                @pl.when(b < _NBLK - 1)
                def _():
                    next_dmas()
            @pl.when((b > 0) & (node0 - _GPB < _N))
            def _():
                pltpu.make_async_copy(
                    outb, out_hbm.at[pl.ds(node0 - _GPB, _GPB)],
                    osem).wait()

            def node(n, c2):
                nb = n * _DEG
                xr = [xi[p, n, pl.ds(r * 16, 16)] for r in range(8)]
                bi = plsc.load_gather(invv, [_splat(node0 + n, jnp.int32)])
                bib = bi * beta
                abv = _splat(jnp.abs(beta))
                denv = jnp.zeros((16,), jnp.float32)
                ao = [jnp.zeros((16,), jnp.float32) for _ in range(8)]
                for g in range(2):
                    jidx = colv3[i3, pl.ds(nb + g * 16, 16)]
                    invj = plsc.load_gather(invv, [jidx])
                    for jj in range(16):
                        rr = nb + g * 16 + jj
                        rv = [rows[p, rr, pl.ds(r * 16, 16)]
                              for r in range(8)]
                        acc = xr[0] * rv[0]
                        for r in range(1, 8):
                            acc = acc + xr[r] * rv[r]
                        sv = _splat(jnp.sum(acc)) * _splat(invj[jj]) * bib
                        w = jnp.exp(sv - abv)
                        denv = denv + w
                        for r in range(8):
                            ao[r] = ao[r] + w * rv[r]
                rden = 1.0 / denv
                for r in range(8):
                    outb[n, pl.ds(r * 16, 16)] = ao[r] * rden
                return c2

            lax.fori_loop(0, _GPB, node, 0)

            @pl.when(node0 < _N)
            def _():
                pltpu.async_copy(outb, out_hbm.at[pl.ds(node0, _GPB)], osem)
        return carry

    lax.fori_loop(0, _NBLK // 2, block_pair, 0)
    last0 = wid * _NPW + (_NBLK - 1) * _GPB

    @pl.when(last0 < _N)
    def _():
        pltpu.make_async_copy(outb, out_hbm.at[pl.ds(last0, _GPB)],
                              osem).wait()


def kernel(x, row_id, row_ptr, col_id, beta):
    del row_id, row_ptr
    colp = jnp.zeros((_NPAD * _DEG,), jnp.int32).at[: _N * _DEG].set(col_id)
    col3 = colp.reshape(_NW, _NBLK, _EPB)
    beta16 = jnp.zeros((16,), jnp.float32).at[0].set(beta[0])
    return _agnn(x, col3, beta16)
